# initial kernel scaffold (unmeasured)
import jax
import jax.numpy as jnp
from jax import lax
from jax.experimental import pallas as pl
from jax.experimental.pallas import tpu as pltpu

N_DEV = 4
N_TOK = 1024
D_IN = 256
D_OUT = 512
N_EXP = 16
E_LOCAL = N_EXP // N_DEV
CAP = 51


def kernel(x, router_W, route_idx, expert_W):
    my = lax.axis_index("i")

    route = route_idx[:, 0]
    onehot = route[:, None] == jnp.arange(N_EXP, dtype=jnp.int32)[None, :]
    ohi = onehot.astype(jnp.int32)
    ranks = jnp.cumsum(ohi, axis=0) - ohi
    keep = jnp.logical_and(onehot, ranks < CAP)
    local_keep = lax.dynamic_slice(keep, (0, my * E_LOCAL), (N_TOK, E_LOCAL))

    x_bf = x.astype(jnp.bfloat16)
    w_bf = expert_W.astype(jnp.bfloat16)
    keep_bf = local_keep.astype(jnp.bfloat16)

    def body(x_ref, keep_ref, w_ref, out_ref, comm_ref, send_sems, recv_sems):
        my_pos = lax.axis_index("i")
        left = (my_pos - 1) % N_DEV
        right = (my_pos + 1) % N_DEV

        barrier_sem = pltpu.get_barrier_semaphore()
        for nbr in (left, right):
            pl.semaphore_signal(
                barrier_sem, inc=1,
                device_id=(nbr,), device_id_type=pl.DeviceIdType.MESH,
            )
        pl.semaphore_wait(barrier_sem, 2)

        acc = jnp.zeros((N_TOK, D_OUT), jnp.float32)
        for le in range(E_LOCAL):
            xm = x_ref[:, :] * keep_ref[:, le : le + 1]
            acc = acc + jnp.dot(
                xm, w_ref[le], preferred_element_type=jnp.float32
            )
        comm_ref[0, :, :] = acc.astype(jnp.bfloat16)

        for h in range(N_DEV - 1):
            rdma = pltpu.make_async_remote_copy(
                src_ref=comm_ref.at[h],
                dst_ref=comm_ref.at[h + 1],
                send_sem=send_sems.at[h],
                recv_sem=recv_sems.at[h],
                device_id=(right,),
                device_id_type=pl.DeviceIdType.MESH,
            )
            rdma.start()
            rdma.wait()
            acc = acc + comm_ref[h + 1, :, :].astype(jnp.float32)

        out_ref[:, :] = acc

    return pl.pallas_call(
        body,
        out_shape=jax.ShapeDtypeStruct((N_TOK, D_OUT), jnp.float32),
        in_specs=[pl.BlockSpec(memory_space=pltpu.VMEM)] * 3,
        out_specs=pl.BlockSpec(memory_space=pltpu.VMEM),
        scratch_shapes=[
            pltpu.VMEM((N_DEV, N_TOK, D_OUT), jnp.bfloat16),
            pltpu.SemaphoreType.DMA((N_DEV - 1,)),
            pltpu.SemaphoreType.DMA((N_DEV - 1,)),
        ],
        compiler_params=pltpu.CompilerParams(collective_id=0),
    )(x_bf, keep_bf, w_bf)


# baseline (device time: 53559 ns/iter reference)
import jax
import jax.numpy as jnp
from jax import lax
from jax.experimental import pallas as pl
from jax.experimental.pallas import tpu as pltpu

N_DEV = 4
N_TOK = 1024
D_IN = 256
D_OUT = 512
N_EXP = 16
E_LOCAL = N_EXP // N_DEV
CAP = 51


def kernel(x, router_W, route_idx, expert_W):
    my = lax.axis_index("i")

    route = route_idx[:, 0]
    onehot = route[:, None] == jnp.arange(N_EXP, dtype=jnp.int32)[None, :]
    ohi = onehot.astype(jnp.int32)
    ranks = jnp.cumsum(ohi, axis=0) - ohi
    keep = jnp.logical_and(onehot, ranks < CAP)
    local_keep = lax.dynamic_slice(keep, (0, my * E_LOCAL), (N_TOK, E_LOCAL))

    x_bf = x.astype(jnp.bfloat16)
    w_bf = expert_W.astype(jnp.bfloat16)
    keep_bf = local_keep.astype(jnp.bfloat16)

    def body(x_ref, keep_ref, w_ref, out_ref, comm_ref, send_sems, recv_sems):
        my_pos = lax.axis_index("i")
        left = (my_pos - 1) % N_DEV
        right = (my_pos + 1) % N_DEV

        barrier_sem = pltpu.get_barrier_semaphore()
        for nbr in (left, right):
            pl.semaphore_signal(
                barrier_sem, inc=1,
                device_id=(nbr,), device_id_type=pl.DeviceIdType.MESH,
            )
        pl.semaphore_wait(barrier_sem, 2)

        out_ref[:, :] = jnp.zeros((N_TOK, D_OUT), jnp.float32)
        for le in range(E_LOCAL):
            xm = x_ref[:, :] * keep_ref[:, le : le + 1]
            out_ref[:, :] += jnp.dot(
                xm, w_ref[le], preferred_element_type=jnp.float32
            )
        comm_ref[0, :, :] = out_ref[:, :].astype(jnp.bfloat16)

        for h in range(N_DEV - 1):
            rdma = pltpu.make_async_remote_copy(
                src_ref=comm_ref.at[h],
                dst_ref=comm_ref.at[h + 1],
                send_sem=send_sems.at[h],
                recv_sem=recv_sems.at[h],
                device_id=(right,),
                device_id_type=pl.DeviceIdType.MESH,
            )
            rdma.start()
            rdma.wait()
            out_ref[:, :] += comm_ref[h + 1, :, :].astype(jnp.float32)

    return pl.pallas_call(
        body,
        out_shape=jax.ShapeDtypeStruct((N_TOK, D_OUT), jnp.float32),
        in_specs=[pl.BlockSpec(memory_space=pltpu.VMEM)] * 3,
        out_specs=pl.BlockSpec(memory_space=pltpu.VMEM),
        scratch_shapes=[
            pltpu.VMEM((N_DEV, N_TOK, D_OUT), jnp.bfloat16),
            pltpu.SemaphoreType.DMA((N_DEV - 1,)),
            pltpu.SemaphoreType.DMA((N_DEV - 1,)),
        ],
        compiler_params=pltpu.CompilerParams(collective_id=0),
    )(x_bf, keep_bf, w_bf)


# device time: 40216 ns/iter; 1.3318x vs baseline; 1.3318x over previous
import jax
import jax.numpy as jnp
from jax import lax
from jax.experimental import pallas as pl
from jax.experimental.pallas import tpu as pltpu

N_DEV = 4
N_TOK = 1024
D_IN = 256
D_OUT = 512
N_EXP = 16
E_LOCAL = N_EXP // N_DEV
CAP = 51


def kernel(x, router_W, route_idx, expert_W):
    my = lax.axis_index("i")

    route = route_idx[:, 0]
    onehot = route[:, None] == jnp.arange(N_EXP, dtype=jnp.int32)[None, :]
    ohi = onehot.astype(jnp.int32)
    ranks = jnp.cumsum(ohi, axis=0) - ohi
    keep = jnp.logical_and(onehot, ranks < CAP)
    local_keep = lax.dynamic_slice(keep, (0, my * E_LOCAL), (N_TOK, E_LOCAL))

    x_bf = x.astype(jnp.bfloat16)
    w_bf = expert_W.astype(jnp.bfloat16)
    keep_bf = local_keep.astype(jnp.bfloat16)

    def body(x_ref, keep_ref, w_ref, out_ref, comm_ref, send_sems, recv_sems):
        my_pos = lax.axis_index("i")
        partner1 = my_pos ^ 1
        partner2 = 3 - my_pos

        barrier_sem = pltpu.get_barrier_semaphore()
        for nbr in (partner1, partner2):
            pl.semaphore_signal(
                barrier_sem, inc=1,
                device_id=(nbr,), device_id_type=pl.DeviceIdType.MESH,
            )
        pl.semaphore_wait(barrier_sem, 2)

        out_ref[:, :] = jnp.zeros((N_TOK, D_OUT), jnp.float32)
        for le in range(E_LOCAL):
            xm = x_ref[:, :] * keep_ref[:, le : le + 1]
            out_ref[:, :] += jnp.dot(
                xm, w_ref[le], preferred_element_type=jnp.float32
            )

        comm_ref[0, :, :] = out_ref[:, :].astype(jnp.bfloat16)
        for s, partner in ((0, partner1), (2, partner2)):
            rdma = pltpu.make_async_remote_copy(
                src_ref=comm_ref.at[s],
                dst_ref=comm_ref.at[s + 1],
                send_sem=send_sems.at[s // 2],
                recv_sem=recv_sems.at[s // 2],
                device_id=(partner,),
                device_id_type=pl.DeviceIdType.MESH,
            )
            rdma.start()
            rdma.wait()
            out_ref[:, :] += comm_ref[s + 1, :, :].astype(jnp.float32)
            if s == 0:
                comm_ref[2, :, :] = out_ref[:, :].astype(jnp.bfloat16)

    return pl.pallas_call(
        body,
        out_shape=jax.ShapeDtypeStruct((N_TOK, D_OUT), jnp.float32),
        in_specs=[pl.BlockSpec(memory_space=pltpu.VMEM)] * 3,
        out_specs=pl.BlockSpec(memory_space=pltpu.VMEM),
        scratch_shapes=[
            pltpu.VMEM((4, N_TOK, D_OUT), jnp.bfloat16),
            pltpu.SemaphoreType.DMA((2,)),
            pltpu.SemaphoreType.DMA((2,)),
        ],
        compiler_params=pltpu.CompilerParams(collective_id=0),
    )(x_bf, keep_bf, w_bf)


# device time: 36980 ns/iter; 1.4483x vs baseline; 1.0875x over previous
import jax
import jax.numpy as jnp
from jax import lax
from jax.experimental import pallas as pl
from jax.experimental.pallas import tpu as pltpu

N_DEV = 4
N_TOK = 1024
D_IN = 256
D_OUT = 512
N_EXP = 16
E_LOCAL = N_EXP // N_DEV
CAP = 51
SLOTS_PER_EXP = 64
C_ROWS = E_LOCAL * SLOTS_PER_EXP


def kernel(x, router_W, route_idx, expert_W):
    my = lax.axis_index("i")

    route = route_idx[:, 0]
    oh = route[:, None] == jnp.arange(N_EXP, dtype=jnp.int32)[None, :]
    ohi = oh.astype(jnp.int32)
    rank = jnp.cumsum(ohi, axis=0) - ohi
    rank_tok = jnp.take_along_axis(rank, route[:, None], axis=1)[:, 0]
    kept = rank_tok < CAP
    gslot = (route // E_LOCAL) * C_ROWS + (route % E_LOCAL) * SLOTS_PER_EXP
    gslot = jnp.where(kept, gslot + rank_tok, -1)

    P = (gslot[:, None] == jnp.arange(N_DEV * C_ROWS, dtype=jnp.int32)[None, :])
    P_bf = P.astype(jnp.bfloat16)
    G = (jnp.arange(C_ROWS, dtype=jnp.int32)[:, None] + my * C_ROWS) == gslot[None, :]
    G_bf = G.astype(jnp.bfloat16)

    x_bf = x.astype(jnp.bfloat16)
    w_bf = expert_W.astype(jnp.bfloat16)

    def body(x_ref, g_ref, p_ref, w_ref, out_ref, comm_ref, send_sems, recv_sems):
        my_pos = lax.axis_index("i")
        partner1 = my_pos ^ 1
        partner2 = 3 - my_pos
        my_base = my_pos * C_ROWS
        pair_base = (my_pos // 2) * (2 * C_ROWS)
        other_base = (2 * C_ROWS) - pair_base

        barrier_sem = pltpu.get_barrier_semaphore()
        for nbr in (partner1, partner2):
            pl.semaphore_signal(
                barrier_sem, inc=1,
                device_id=(nbr,), device_id_type=pl.DeviceIdType.MESH,
            )
        pl.semaphore_wait(barrier_sem, 2)

        xc = jnp.dot(g_ref[:, :], x_ref[:, :],
                     preferred_element_type=jnp.float32).astype(jnp.bfloat16)
        for le in range(E_LOCAL):
            comm_ref[pl.ds(my_base + le * SLOTS_PER_EXP, SLOTS_PER_EXP), :] = (
                jnp.dot(xc[le * SLOTS_PER_EXP:(le + 1) * SLOTS_PER_EXP, :],
                        w_ref[le],
                        preferred_element_type=jnp.float32).astype(jnp.bfloat16)
            )

        rdma1 = pltpu.make_async_remote_copy(
            src_ref=comm_ref.at[pl.ds(my_base, C_ROWS)],
            dst_ref=comm_ref.at[pl.ds(my_base, C_ROWS)],
            send_sem=send_sems.at[0],
            recv_sem=recv_sems.at[0],
            device_id=(partner1,),
            device_id_type=pl.DeviceIdType.MESH,
        )
        rdma1.start()
        rdma1.wait()

        rdma2 = pltpu.make_async_remote_copy(
            src_ref=comm_ref.at[pl.ds(pair_base, 2 * C_ROWS)],
            dst_ref=comm_ref.at[pl.ds(pair_base, 2 * C_ROWS)],
            send_sem=send_sems.at[1],
            recv_sem=recv_sems.at[1],
            device_id=(partner2,),
            device_id_type=pl.DeviceIdType.MESH,
        )
        rdma2.start()
        out_ref[:, :] = jnp.dot(
            p_ref[:, pl.ds(pair_base, 2 * C_ROWS)],
            comm_ref[pl.ds(pair_base, 2 * C_ROWS), :],
            preferred_element_type=jnp.float32,
        )
        rdma2.wait()
        out_ref[:, :] += jnp.dot(
            p_ref[:, pl.ds(other_base, 2 * C_ROWS)],
            comm_ref[pl.ds(other_base, 2 * C_ROWS), :],
            preferred_element_type=jnp.float32,
        )

    return pl.pallas_call(
        body,
        out_shape=jax.ShapeDtypeStruct((N_TOK, D_OUT), jnp.float32),
        in_specs=[pl.BlockSpec(memory_space=pltpu.VMEM)] * 4,
        out_specs=pl.BlockSpec(memory_space=pltpu.VMEM),
        scratch_shapes=[
            pltpu.VMEM((N_DEV * C_ROWS, D_OUT), jnp.bfloat16),
            pltpu.SemaphoreType.DMA((2,)),
            pltpu.SemaphoreType.DMA((2,)),
        ],
        compiler_params=pltpu.CompilerParams(collective_id=0),
    )(x_bf, G_bf, P_bf, w_bf)


# device time: 21730 ns/iter; 2.4647x vs baseline; 1.7018x over previous
import jax
import jax.numpy as jnp
from jax import lax
from jax.experimental import pallas as pl
from jax.experimental.pallas import tpu as pltpu

N_DEV = 4
N_TOK = 1024
D_IN = 256
D_OUT = 512
N_EXP = 16
E_LOCAL = N_EXP // N_DEV
CAP = 51
SLOTS_PER_EXP = 64
C_ROWS = E_LOCAL * SLOTS_PER_EXP
G_ROWS = N_DEV * C_ROWS


def kernel(x, router_W, route_idx, expert_W):
    x_bf = x.astype(jnp.bfloat16)
    w_bf = expert_W.astype(jnp.bfloat16)

    def body(x_ref, route_ref, w_ref, out_ref, p_ref, comm_ref,
             send_sems, recv_sems):
        my_pos = lax.axis_index("i")
        partner1 = my_pos ^ 1
        partner2 = 3 - my_pos
        my_base = my_pos * C_ROWS
        pair_base = (my_pos // 2) * (2 * C_ROWS)
        other_base = (2 * C_ROWS) - pair_base

        barrier_sem = pltpu.get_barrier_semaphore()
        for nbr in (partner1, partner2):
            pl.semaphore_signal(
                barrier_sem, inc=1,
                device_id=(nbr,), device_id_type=pl.DeviceIdType.MESH,
            )

        e_tok = route_ref[:, :]
        oh = (e_tok == lax.broadcasted_iota(jnp.int32, (N_TOK, N_EXP), 1))
        oh_bf = oh.astype(jnp.bfloat16)
        tri = (lax.broadcasted_iota(jnp.int32, (N_TOK, N_TOK), 0)
               > lax.broadcasted_iota(jnp.int32, (N_TOK, N_TOK), 1))
        ranks = jnp.dot(tri.astype(jnp.bfloat16), oh_bf,
                        preferred_element_type=jnp.float32)
        rank_tok = jnp.sum(ranks * oh_bf.astype(jnp.float32),
                           axis=1, keepdims=True)
        kept = rank_tok < float(CAP)
        gslot = ((e_tok // E_LOCAL) * C_ROWS
                 + (e_tok % E_LOCAL) * SLOTS_PER_EXP
                 + rank_tok.astype(jnp.int32))
        gslot = jnp.where(kept, gslot, -1)
        p_ref[:, :] = (
            gslot == lax.broadcasted_iota(jnp.int32, (N_TOK, G_ROWS), 1)
        ).astype(jnp.bfloat16)

        xc = lax.dot_general(
            p_ref[:, pl.ds(my_base, C_ROWS)], x_ref[:, :],
            ((( 0,), (0,)), ((), ())),
            preferred_element_type=jnp.float32,
        ).astype(jnp.bfloat16)
        for le in range(E_LOCAL):
            comm_ref[pl.ds(my_base + le * SLOTS_PER_EXP, SLOTS_PER_EXP), :] = (
                jnp.dot(xc[le * SLOTS_PER_EXP:(le + 1) * SLOTS_PER_EXP, :],
                        w_ref[le],
                        preferred_element_type=jnp.float32).astype(jnp.bfloat16)
            )

        pl.semaphore_wait(barrier_sem, 2)

        rdma1 = pltpu.make_async_remote_copy(
            src_ref=comm_ref.at[pl.ds(my_base, C_ROWS)],
            dst_ref=comm_ref.at[pl.ds(my_base, C_ROWS)],
            send_sem=send_sems.at[0],
            recv_sem=recv_sems.at[0],
            device_id=(partner1,),
            device_id_type=pl.DeviceIdType.MESH,
        )
        rdma1.start()
        rdma1.wait()

        rdma2 = pltpu.make_async_remote_copy(
            src_ref=comm_ref.at[pl.ds(pair_base, 2 * C_ROWS)],
            dst_ref=comm_ref.at[pl.ds(pair_base, 2 * C_ROWS)],
            send_sem=send_sems.at[1],
            recv_sem=recv_sems.at[1],
            device_id=(partner2,),
            device_id_type=pl.DeviceIdType.MESH,
        )
        rdma2.start()
        out_ref[:, :] = jnp.dot(
            p_ref[:, pl.ds(pair_base, 2 * C_ROWS)],
            comm_ref[pl.ds(pair_base, 2 * C_ROWS), :],
            preferred_element_type=jnp.float32,
        )
        rdma2.wait()
        out_ref[:, :] += jnp.dot(
            p_ref[:, pl.ds(other_base, 2 * C_ROWS)],
            comm_ref[pl.ds(other_base, 2 * C_ROWS), :],
            preferred_element_type=jnp.float32,
        )

    return pl.pallas_call(
        body,
        out_shape=jax.ShapeDtypeStruct((N_TOK, D_OUT), jnp.float32),
        in_specs=[pl.BlockSpec(memory_space=pltpu.VMEM)] * 3,
        out_specs=pl.BlockSpec(memory_space=pltpu.VMEM),
        scratch_shapes=[
            pltpu.VMEM((N_TOK, G_ROWS), jnp.bfloat16),
            pltpu.VMEM((G_ROWS, D_OUT), jnp.bfloat16),
            pltpu.SemaphoreType.DMA((2,)),
            pltpu.SemaphoreType.DMA((2,)),
        ],
        compiler_params=pltpu.CompilerParams(collective_id=0),
    )(x_bf, route_idx, w_bf)


# device time: 18692 ns/iter; 2.8653x vs baseline; 1.1625x over previous
import jax
import jax.numpy as jnp
from jax import lax
from jax.experimental import pallas as pl
from jax.experimental.pallas import tpu as pltpu

N_DEV = 4
N_TOK = 1024
D_IN = 256
D_OUT = 512
N_EXP = 16
E_LOCAL = N_EXP // N_DEV
CAP = 51
SLOTS_PER_EXP = 64
C_ROWS = E_LOCAL * SLOTS_PER_EXP
G_ROWS = N_DEV * C_ROWS


def kernel(x, router_W, route_idx, expert_W):
    x_bf = x.astype(jnp.bfloat16)
    w_bf = expert_W.astype(jnp.bfloat16)

    def body(x_ref, route_ref, w_ref, out_ref, p_ref, comm_ref,
             send_sems, recv_sems):
        my_pos = lax.axis_index("i")
        partner1 = my_pos ^ 1
        partner2 = 3 - my_pos
        my_base = my_pos * C_ROWS
        p1_base = partner1 * C_ROWS
        p2_base = partner2 * C_ROWS
        diag_base = (my_pos ^ 2) * C_ROWS
        pair_base = (my_pos // 2) * (2 * C_ROWS)

        barrier_sem = pltpu.get_barrier_semaphore()
        for nbr in (partner1, partner2):
            pl.semaphore_signal(
                barrier_sem, inc=1,
                device_id=(nbr,), device_id_type=pl.DeviceIdType.MESH,
            )

        e_tok = route_ref[:, :]
        oh = (e_tok == lax.broadcasted_iota(jnp.int32, (N_TOK, N_EXP), 1))
        oh_bf = oh.astype(jnp.bfloat16)
        tri = (lax.broadcasted_iota(jnp.int32, (N_TOK, N_TOK), 0)
               > lax.broadcasted_iota(jnp.int32, (N_TOK, N_TOK), 1))
        ranks = jnp.dot(tri.astype(jnp.bfloat16), oh_bf,
                        preferred_element_type=jnp.float32)
        rank_tok = jnp.sum(ranks * oh_bf.astype(jnp.float32),
                           axis=1, keepdims=True)
        kept = rank_tok < float(CAP)
        gslot = ((e_tok // E_LOCAL) * C_ROWS
                 + (e_tok % E_LOCAL) * SLOTS_PER_EXP
                 + rank_tok.astype(jnp.int32))
        gslot = jnp.where(kept, gslot, -1)
        p_ref[:, pl.ds(my_base, C_ROWS)] = (
            gslot == (lax.broadcasted_iota(jnp.int32, (N_TOK, C_ROWS), 1)
                      + my_base)
        ).astype(jnp.bfloat16)

        xc = lax.dot_general(
            p_ref[:, pl.ds(my_base, C_ROWS)], x_ref[:, :],
            ((( 0,), (0,)), ((), ())),
            preferred_element_type=jnp.float32,
        ).astype(jnp.bfloat16)
        for le in range(E_LOCAL):
            comm_ref[pl.ds(my_base + le * SLOTS_PER_EXP, SLOTS_PER_EXP), :] = (
                jnp.dot(xc[le * SLOTS_PER_EXP:(le + 1) * SLOTS_PER_EXP, :],
                        w_ref[le],
                        preferred_element_type=jnp.float32).astype(jnp.bfloat16)
            )

        pl.semaphore_wait(barrier_sem, 2)

        rdma_a = pltpu.make_async_remote_copy(
            src_ref=comm_ref.at[pl.ds(my_base, C_ROWS)],
            dst_ref=comm_ref.at[pl.ds(my_base, C_ROWS)],
            send_sem=send_sems.at[0],
            recv_sem=recv_sems.at[0],
            device_id=(partner1,),
            device_id_type=pl.DeviceIdType.MESH,
        )
        rdma_b = pltpu.make_async_remote_copy(
            src_ref=comm_ref.at[pl.ds(my_base, C_ROWS)],
            dst_ref=comm_ref.at[pl.ds(my_base, C_ROWS)],
            send_sem=send_sems.at[1],
            recv_sem=recv_sems.at[1],
            device_id=(partner2,),
            device_id_type=pl.DeviceIdType.MESH,
        )
        rdma_a.start()
        rdma_b.start()

        p_ref[:, :] = (
            gslot == lax.broadcasted_iota(jnp.int32, (N_TOK, G_ROWS), 1)
        ).astype(jnp.bfloat16)

        rdma_a.wait_recv()
        rdma_c = pltpu.make_async_remote_copy(
            src_ref=comm_ref.at[pl.ds(p1_base, C_ROWS)],
            dst_ref=comm_ref.at[pl.ds(p1_base, C_ROWS)],
            send_sem=send_sems.at[2],
            recv_sem=recv_sems.at[2],
            device_id=(partner2,),
            device_id_type=pl.DeviceIdType.MESH,
        )
        rdma_c.start()

        out_ref[:, :] = jnp.dot(
            p_ref[:, pl.ds(pair_base, 2 * C_ROWS)],
            comm_ref[pl.ds(pair_base, 2 * C_ROWS), :],
            preferred_element_type=jnp.float32,
        )
        rdma_b.wait_recv()
        out_ref[:, :] += jnp.dot(
            p_ref[:, pl.ds(p2_base, C_ROWS)],
            comm_ref[pl.ds(p2_base, C_ROWS), :],
            preferred_element_type=jnp.float32,
        )
        rdma_c.wait_recv()
        out_ref[:, :] += jnp.dot(
            p_ref[:, pl.ds(diag_base, C_ROWS)],
            comm_ref[pl.ds(diag_base, C_ROWS), :],
            preferred_element_type=jnp.float32,
        )

        rdma_a.wait_send()
        rdma_b.wait_send()
        rdma_c.wait_send()

    return pl.pallas_call(
        body,
        out_shape=jax.ShapeDtypeStruct((N_TOK, D_OUT), jnp.float32),
        in_specs=[pl.BlockSpec(memory_space=pltpu.VMEM)] * 3,
        out_specs=pl.BlockSpec(memory_space=pltpu.VMEM),
        scratch_shapes=[
            pltpu.VMEM((N_TOK, G_ROWS), jnp.bfloat16),
            pltpu.VMEM((G_ROWS, D_OUT), jnp.bfloat16),
            pltpu.SemaphoreType.DMA((3,)),
            pltpu.SemaphoreType.DMA((3,)),
        ],
        compiler_params=pltpu.CompilerParams(collective_id=0),
    )(x_bf, route_idx, w_bf)


# device time: 17984 ns/iter; 2.9781x vs baseline; 1.0394x over previous
import jax
import jax.numpy as jnp
from jax import lax
from jax.experimental import pallas as pl
from jax.experimental.pallas import tpu as pltpu

N_DEV = 4
N_TOK = 1024
D_IN = 256
D_OUT = 512
N_EXP = 16
E_LOCAL = N_EXP // N_DEV
CAP = 51
SLOTS_PER_EXP = 64
C_ROWS = E_LOCAL * SLOTS_PER_EXP
G_ROWS = N_DEV * C_ROWS


def kernel(x, router_W, route_idx, expert_W):
    x_bf = x.astype(jnp.bfloat16)
    w_bf = expert_W.astype(jnp.bfloat16)

    def body(x_ref, route_ref, w_ref, out_ref, p_ref, comm_ref,
             send_sems, recv_sems):
        my_pos = lax.axis_index("i")
        partner1 = my_pos ^ 1
        partner2 = 3 - my_pos
        diag = my_pos ^ 2
        my_base = my_pos * C_ROWS
        p1_base = partner1 * C_ROWS
        p2_base = partner2 * C_ROWS
        diag_base = diag * C_ROWS

        barrier_sem = pltpu.get_barrier_semaphore()
        for nbr in (partner1, partner2, diag):
            pl.semaphore_signal(
                barrier_sem, inc=1,
                device_id=(nbr,), device_id_type=pl.DeviceIdType.MESH,
            )

        e_tok = route_ref[:, :]
        oh = (e_tok == lax.broadcasted_iota(jnp.int32, (N_TOK, N_EXP), 1))
        oh_bf = oh.astype(jnp.bfloat16)
        tri = (lax.broadcasted_iota(jnp.int32, (N_TOK, N_TOK), 0)
               > lax.broadcasted_iota(jnp.int32, (N_TOK, N_TOK), 1))
        ranks = jnp.dot(tri.astype(jnp.bfloat16), oh_bf,
                        preferred_element_type=jnp.float32)
        rank_tok = jnp.sum(ranks * oh_bf.astype(jnp.float32),
                           axis=1, keepdims=True)
        kept = rank_tok < float(CAP)
        gslot = ((e_tok // E_LOCAL) * C_ROWS
                 + (e_tok % E_LOCAL) * SLOTS_PER_EXP
                 + rank_tok.astype(jnp.int32))
        gslot = jnp.where(kept, gslot, -1)
        p_ref[:, pl.ds(my_base, C_ROWS)] = (
            gslot == (lax.broadcasted_iota(jnp.int32, (N_TOK, C_ROWS), 1)
                      + my_base)
        ).astype(jnp.bfloat16)

        xc = lax.dot_general(
            p_ref[:, pl.ds(my_base, C_ROWS)], x_ref[:, :],
            ((( 0,), (0,)), ((), ())),
            preferred_element_type=jnp.float32,
        ).astype(jnp.bfloat16)
        for le in range(E_LOCAL):
            comm_ref[pl.ds(my_base + le * SLOTS_PER_EXP, SLOTS_PER_EXP), :] = (
                jnp.dot(xc[le * SLOTS_PER_EXP:(le + 1) * SLOTS_PER_EXP, :],
                        w_ref[le],
                        preferred_element_type=jnp.float32).astype(jnp.bfloat16)
            )

        pl.semaphore_wait(barrier_sem, 3)

        rdmas = []
        for k, peer in enumerate((partner1, partner2, diag)):
            r = pltpu.make_async_remote_copy(
                src_ref=comm_ref.at[pl.ds(my_base, C_ROWS)],
                dst_ref=comm_ref.at[pl.ds(my_base, C_ROWS)],
                send_sem=send_sems.at[k],
                recv_sem=recv_sems.at[k],
                device_id=(peer,),
                device_id_type=pl.DeviceIdType.MESH,
            )
            r.start()
            rdmas.append(r)

        p_ref[:, :] = (
            gslot == lax.broadcasted_iota(jnp.int32, (N_TOK, G_ROWS), 1)
        ).astype(jnp.bfloat16)
        out_ref[:, :] = jnp.dot(
            p_ref[:, pl.ds(my_base, C_ROWS)],
            comm_ref[pl.ds(my_base, C_ROWS), :],
            preferred_element_type=jnp.float32,
        )

        for r, base in zip(rdmas, (p1_base, p2_base, diag_base)):
            r.wait_recv()
            out_ref[:, :] += jnp.dot(
                p_ref[:, pl.ds(base, C_ROWS)],
                comm_ref[pl.ds(base, C_ROWS), :],
                preferred_element_type=jnp.float32,
            )

        for r in rdmas:
            r.wait_send()

    return pl.pallas_call(
        body,
        out_shape=jax.ShapeDtypeStruct((N_TOK, D_OUT), jnp.float32),
        in_specs=[pl.BlockSpec(memory_space=pltpu.VMEM)] * 3,
        out_specs=pl.BlockSpec(memory_space=pltpu.VMEM),
        scratch_shapes=[
            pltpu.VMEM((N_TOK, G_ROWS), jnp.bfloat16),
            pltpu.VMEM((G_ROWS, D_OUT), jnp.bfloat16),
            pltpu.SemaphoreType.DMA((3,)),
            pltpu.SemaphoreType.DMA((3,)),
        ],
        compiler_params=pltpu.CompilerParams(collective_id=0),
    )(x_bf, route_idx, w_bf)
